# baseline (speedup 1.0000x reference)
import functools

import jax
import jax.numpy as jnp
from jax import lax
from jax.experimental import pallas as pl
from jax.experimental.pallas import tpu as pltpu
from jax.experimental.pallas import tpu_sc as plsc

K_TOP = 32
LANES = 16
CHUNK = 32


def _ce(v, i, l, asc):
    a, b = v[i], v[l]
    if asc:
        v[i] = jnp.minimum(a, b)
        v[l] = jnp.maximum(a, b)
    else:
        v[i] = jnp.maximum(a, b)
        v[l] = jnp.minimum(a, b)


def _batcher_comparators(n):
    result = []

    def oddeven_merge(lo, n2, r):
        m = r * 2
        if m < n2:
            oddeven_merge(lo, n2, m)
            oddeven_merge(lo + r, n2, m)
            for i in range(lo + r, lo + n2 - r, m):
                result.append((i, i + r))
        else:
            result.append((lo, lo + r))

    def sort(lo, n2):
        if n2 > 1:
            m = n2 // 2
            sort(lo, m)
            sort(lo + m, m)
            oddeven_merge(lo, n2, 1)

    sort(0, n)
    return result


def _batcher_sort_desc(v):
    for i, l in _batcher_comparators(len(v)):
        _ce(v, i, l, asc=False)


def _bitonic_merge_desc(v):
    n = len(v)
    j = n // 2
    while j >= 1:
        for i in range(n):
            l = i ^ j
            if l > i:
                _ce(v, i, l, asc=False)
        j //= 2


def _make_sc_kernel(B, L, D):
    NC, NS = 2, 16
    NW = NC * NS
    d_tiles = D // LANES
    n_tiles = B * d_tiles
    assert n_tiles % NW == 0
    tiles_per_w = n_tiles // NW
    HL = L // 2
    half_chunks = HL // CHUNK

    mesh = plsc.VectorSubcoreMesh(
        core_axis_name="c", subcore_axis_name="s", num_cores=NC, num_subcores=NS
    )

    @functools.partial(
        pl.kernel,
        out_type=jax.ShapeDtypeStruct((B, K_TOP * D), jnp.float32),
        mesh=mesh,
        scratch_types=[
            pltpu.VMEM((L // 2, LANES), jnp.float32),
            pltpu.VMEM((L // 2, LANES), jnp.float32),
            pltpu.VMEM((K_TOP * LANES,), jnp.float32),
            pltpu.SemaphoreType.DMA,
            pltpu.SemaphoreType.DMA,
        ],
        compiler_params=pltpu.CompilerParams(
            use_tc_tiling_on_sc=False, needs_layout_passes=False
        ),
    )
    def topk_kernel(x_hbm, out_hbm, buf0, buf1, obuf, sem0, sem1):
        wid = lax.axis_index("s") * NC + lax.axis_index("c")
        lane = lax.iota(jnp.int32, LANES)

        def coords(tile):
            return tile // d_tiles, (tile % d_tiles) * LANES

        def half_src(b, d0, h):
            return x_hbm.at[b, pl.ds(h * HL, HL), pl.ds(d0, LANES)]

        def make_chunk_body(buf):
            def chunk_body(c, acc):
                acc = list(acc)
                base = c * CHUNK
                ch = [buf[base + i] for i in range(CHUNK)]
                _batcher_sort_desc(ch)
                m = [
                    jnp.maximum(acc[i], ch[K_TOP - 1 - i])
                    if i >= K_TOP - CHUNK
                    else acc[i]
                    for i in range(K_TOP)
                ]
                _bitonic_merge_desc(m)
                return tuple(m)

            return chunk_body

        neg_inf = jnp.full((LANES,), -jnp.inf, jnp.float32)
        acc0 = (neg_inf,) * K_TOP

        b_p, d0_p = coords(wid * tiles_per_w)
        pltpu.make_async_copy(half_src(b_p, d0_p, 0), buf0, sem0).start()

        def tile_body(t, _):
            tile = wid * tiles_per_w + t
            b, d0 = coords(tile)
            pltpu.make_async_copy(half_src(b, d0, 1), buf1, sem1).start()
            pltpu.make_async_copy(half_src(b, d0, 0), buf0, sem0).wait()
            acc = lax.fori_loop(
                0, half_chunks, make_chunk_body(buf0), acc0, unroll=False
            )

            @pl.when(t + 1 < tiles_per_w)
            def _():
                b_n, d0_n = coords(tile + 1)
                pltpu.make_async_copy(half_src(b_n, d0_n, 0), buf0, sem0).start()

            pltpu.make_async_copy(half_src(b, d0, 1), buf1, sem1).wait()
            acc = lax.fori_loop(
                0, half_chunks, make_chunk_body(buf1), acc, unroll=False
            )

            for k in range(K_TOP):
                plsc.store_scatter(obuf, [lane * K_TOP + k], acc[k])
            pltpu.sync_copy(obuf, out_hbm.at[b, pl.ds(d0 * K_TOP, K_TOP * LANES)])
            return 0

        lax.fori_loop(0, tiles_per_w, tile_body, 0, unroll=False)

    return topk_kernel


def kernel(x):
    B, L, D = x.shape
    return _make_sc_kernel(B, L, D)(x)


# speedup vs baseline: 1.0654x; 1.0654x over previous
import functools

import jax
import jax.numpy as jnp
from jax import lax
from jax.experimental import pallas as pl
from jax.experimental.pallas import tpu as pltpu
from jax.experimental.pallas import tpu_sc as plsc

K_TOP = 32
LANES = 16
CHUNK = 16


def _ce(v, i, l, asc):
    a, b = v[i], v[l]
    if asc:
        v[i] = jnp.minimum(a, b)
        v[l] = jnp.maximum(a, b)
    else:
        v[i] = jnp.maximum(a, b)
        v[l] = jnp.minimum(a, b)


def _batcher_comparators(n):
    result = []

    def oddeven_merge(lo, n2, r):
        m = r * 2
        if m < n2:
            oddeven_merge(lo, n2, m)
            oddeven_merge(lo + r, n2, m)
            for i in range(lo + r, lo + n2 - r, m):
                result.append((i, i + r))
        else:
            result.append((lo, lo + r))

    def sort(lo, n2):
        if n2 > 1:
            m = n2 // 2
            sort(lo, m)
            sort(lo + m, m)
            oddeven_merge(lo, n2, 1)

    sort(0, n)
    return result


def _batcher_sort_desc(v):
    for i, l in _batcher_comparators(len(v)):
        _ce(v, i, l, asc=False)


def _bitonic_merge_desc(v):
    n = len(v)
    j = n // 2
    while j >= 1:
        for i in range(n):
            l = i ^ j
            if l > i:
                _ce(v, i, l, asc=False)
        j //= 2


def _make_sc_kernel(B, L, D):
    NC, NS = 2, 16
    NW = NC * NS
    d_tiles = D // LANES
    n_tiles = B * d_tiles
    assert n_tiles % NW == 0
    tiles_per_w = n_tiles // NW
    HL = L // 2
    half_chunks = HL // CHUNK

    mesh = plsc.VectorSubcoreMesh(
        core_axis_name="c", subcore_axis_name="s", num_cores=NC, num_subcores=NS
    )

    @functools.partial(
        pl.kernel,
        out_type=jax.ShapeDtypeStruct((B, K_TOP * D), jnp.float32),
        mesh=mesh,
        scratch_types=[
            pltpu.VMEM((L // 2, LANES), jnp.float32),
            pltpu.VMEM((L // 2, LANES), jnp.float32),
            pltpu.VMEM((K_TOP * LANES,), jnp.float32),
            pltpu.SemaphoreType.DMA,
            pltpu.SemaphoreType.DMA,
        ],
        compiler_params=pltpu.CompilerParams(
            use_tc_tiling_on_sc=False, needs_layout_passes=False
        ),
    )
    def topk_kernel(x_hbm, out_hbm, buf0, buf1, obuf, sem0, sem1):
        wid = lax.axis_index("s") * NC + lax.axis_index("c")
        lane = lax.iota(jnp.int32, LANES)

        def coords(tile):
            return tile // d_tiles, (tile % d_tiles) * LANES

        def half_src(b, d0, h):
            return x_hbm.at[b, pl.ds(h * HL, HL), pl.ds(d0, LANES)]

        def make_chunk_body(buf):
            def chunk_body(c, acc):
                acc = list(acc)
                base = c * CHUNK
                ch = [buf[base + i] for i in range(CHUNK)]
                _batcher_sort_desc(ch)
                m = [
                    jnp.maximum(acc[i], ch[K_TOP - 1 - i])
                    if i >= K_TOP - CHUNK
                    else acc[i]
                    for i in range(K_TOP)
                ]
                _bitonic_merge_desc(m)
                return tuple(m)

            return chunk_body

        neg_inf = jnp.full((LANES,), -jnp.inf, jnp.float32)
        acc0 = (neg_inf,) * K_TOP

        b_p, d0_p = coords(wid * tiles_per_w)
        pltpu.make_async_copy(half_src(b_p, d0_p, 0), buf0, sem0).start()

        def tile_body(t, _):
            tile = wid * tiles_per_w + t
            b, d0 = coords(tile)
            pltpu.make_async_copy(half_src(b, d0, 1), buf1, sem1).start()
            pltpu.make_async_copy(half_src(b, d0, 0), buf0, sem0).wait()
            acc = lax.fori_loop(
                0, half_chunks, make_chunk_body(buf0), acc0, unroll=False
            )

            @pl.when(t + 1 < tiles_per_w)
            def _():
                b_n, d0_n = coords(tile + 1)
                pltpu.make_async_copy(half_src(b_n, d0_n, 0), buf0, sem0).start()

            pltpu.make_async_copy(half_src(b, d0, 1), buf1, sem1).wait()
            acc = lax.fori_loop(
                0, half_chunks, make_chunk_body(buf1), acc, unroll=False
            )

            for k in range(K_TOP):
                plsc.store_scatter(obuf, [lane * K_TOP + k], acc[k])
            pltpu.sync_copy(obuf, out_hbm.at[b, pl.ds(d0 * K_TOP, K_TOP * LANES)])
            return 0

        lax.fori_loop(0, tiles_per_w, tile_body, 0, unroll=False)

    return topk_kernel


def kernel(x):
    B, L, D = x.shape
    return _make_sc_kernel(B, L, D)(x)


# speedup vs baseline: 1.5835x; 1.4863x over previous
import functools

import jax
import jax.numpy as jnp
from jax import lax
from jax.experimental import pallas as pl
from jax.experimental.pallas import tpu as pltpu
from jax.experimental.pallas import tpu_sc as plsc

K_TOP = 32
LANES = 16
CHUNK = 16


def _ce(v, i, l, asc):
    a, b = v[i], v[l]
    if asc:
        v[i] = jnp.minimum(a, b)
        v[l] = jnp.maximum(a, b)
    else:
        v[i] = jnp.maximum(a, b)
        v[l] = jnp.minimum(a, b)


def _batcher_comparators(n):
    result = []

    def oddeven_merge(lo, n2, r):
        m = r * 2
        if m < n2:
            oddeven_merge(lo, n2, m)
            oddeven_merge(lo + r, n2, m)
            for i in range(lo + r, lo + n2 - r, m):
                result.append((i, i + r))
        else:
            result.append((lo, lo + r))

    def sort(lo, n2):
        if n2 > 1:
            m = n2 // 2
            sort(lo, m)
            sort(lo + m, m)
            oddeven_merge(lo, n2, 1)

    sort(0, n)
    return result


def _batcher_sort_desc(v):
    for i, l in _batcher_comparators(len(v)):
        _ce(v, i, l, asc=False)


def _bitonic_merge_desc(v):
    n = len(v)
    j = n // 2
    while j >= 1:
        for i in range(n):
            l = i ^ j
            if l > i:
                _ce(v, i, l, asc=False)
        j //= 2


def _make_sc_kernel(B, L, D, d_lo=0):
    NC, NS = 2, 16
    NW = NC * NS
    D_sc = D - d_lo
    d_tiles = D_sc // LANES
    n_tiles = B * d_tiles
    assert n_tiles % NW == 0
    tiles_per_w = n_tiles // NW
    HL = L // 2
    half_chunks = HL // CHUNK

    mesh = plsc.VectorSubcoreMesh(
        core_axis_name="c", subcore_axis_name="s", num_cores=NC, num_subcores=NS
    )

    @functools.partial(
        pl.kernel,
        out_type=jax.ShapeDtypeStruct((B, K_TOP * D_sc), jnp.float32),
        mesh=mesh,
        scratch_types=[
            pltpu.VMEM((L // 2, LANES), jnp.float32),
            pltpu.VMEM((L // 2, LANES), jnp.float32),
            pltpu.VMEM((K_TOP * LANES,), jnp.float32),
            pltpu.SemaphoreType.DMA,
            pltpu.SemaphoreType.DMA,
        ],
        compiler_params=pltpu.CompilerParams(
            use_tc_tiling_on_sc=False, needs_layout_passes=False
        ),
    )
    def topk_kernel(x_hbm, out_hbm, buf0, buf1, obuf, sem0, sem1):
        wid = lax.axis_index("s") * NC + lax.axis_index("c")
        lane = lax.iota(jnp.int32, LANES)

        def coords(tile):
            return tile // d_tiles, d_lo + (tile % d_tiles) * LANES

        def half_src(b, d0, h):
            return x_hbm.at[b, pl.ds(h * HL, HL), pl.ds(d0, LANES)]

        def make_chunk_body(buf):
            def chunk_body(c, acc):
                acc = list(acc)
                base = c * CHUNK
                ch = [buf[base + i] for i in range(CHUNK)]
                _batcher_sort_desc(ch)
                m = [
                    jnp.maximum(acc[i], ch[K_TOP - 1 - i])
                    if i >= K_TOP - CHUNK
                    else acc[i]
                    for i in range(K_TOP)
                ]
                _bitonic_merge_desc(m)
                return tuple(m)

            return chunk_body

        neg_inf = jnp.full((LANES,), -jnp.inf, jnp.float32)
        acc0 = (neg_inf,) * K_TOP

        b_p, d0_p = coords(wid * tiles_per_w)
        pltpu.make_async_copy(half_src(b_p, d0_p, 0), buf0, sem0).start()

        def tile_body(t, _):
            tile = wid * tiles_per_w + t
            b, d0 = coords(tile)
            pltpu.make_async_copy(half_src(b, d0, 1), buf1, sem1).start()
            pltpu.make_async_copy(half_src(b, d0, 0), buf0, sem0).wait()
            acc = lax.fori_loop(
                0, half_chunks, make_chunk_body(buf0), acc0, unroll=False
            )

            @pl.when(t + 1 < tiles_per_w)
            def _():
                b_n, d0_n = coords(tile + 1)
                pltpu.make_async_copy(half_src(b_n, d0_n, 0), buf0, sem0).start()

            pltpu.make_async_copy(half_src(b, d0, 1), buf1, sem1).wait()
            acc = lax.fori_loop(
                0, half_chunks, make_chunk_body(buf1), acc, unroll=False
            )

            for k in range(K_TOP):
                plsc.store_scatter(obuf, [lane * K_TOP + k], acc[k])
            pltpu.sync_copy(
                obuf, out_hbm.at[b, pl.ds((d0 - d_lo) * K_TOP, K_TOP * LANES)]
            )
            return 0

        lax.fori_loop(0, tiles_per_w, tile_body, 0, unroll=False)

    return topk_kernel


def _make_tc_kernel(B, L, d_hi):
    n_dt = d_hi // 128
    depth = 16
    rows_per_chunk = depth * 8
    n_ch = L // rows_per_chunk

    def body(x_ref, o_ref):
        def chunk_body(c, acc):
            acc = list(acc)
            ch = [
                x_ref[0, pl.ds(c * rows_per_chunk + i * 8, 8), :]
                for i in range(depth)
            ]
            _batcher_sort_desc(ch)
            m = [
                jnp.maximum(acc[i], ch[K_TOP - 1 - i])
                if i >= K_TOP - depth
                else acc[i]
                for i in range(K_TOP)
            ]
            _bitonic_merge_desc(m)
            return tuple(m)

        acc0 = (jnp.full((8, 128), -jnp.inf, jnp.float32),) * K_TOP
        acc = list(lax.fori_loop(0, n_ch, chunk_body, acc0, unroll=False))

        for r in (4, 2, 1):
            rolled = [pltpu.roll(a, r, 0) for a in acc]
            m = [
                jnp.maximum(acc[i], rolled[K_TOP - 1 - i]) for i in range(K_TOP)
            ]
            _bitonic_merge_desc(m)
            acc = m

        o_ref[0, 0] = jnp.concatenate([a[0:1] for a in acc], axis=0)

    return pl.pallas_call(
        body,
        grid=(B, n_dt),
        in_specs=[pl.BlockSpec((1, L, 128), lambda b, dt: (b, 0, dt))],
        out_specs=pl.BlockSpec((1, 1, K_TOP, 128), lambda b, dt: (b, dt, 0, 0)),
        out_shape=jax.ShapeDtypeStruct((B, n_dt, K_TOP, 128), jnp.float32),
    )


D_TC = 1024


def kernel(x):
    B, L, D = x.shape
    if D_TC == 0:
        return _make_sc_kernel(B, L, D)(x)
    out_sc = _make_sc_kernel(B, L, D, D_TC)(x)
    out_tc = _make_tc_kernel(B, L, D_TC)(x)
    out_tc = jnp.swapaxes(out_tc, 2, 3).reshape(B, K_TOP * D_TC)
    return jnp.concatenate([out_tc, out_sc], axis=1)
